# Initial kernel scaffold; baseline (speedup 1.0000x reference)
import jax
import jax.numpy as jnp
from jax.experimental import pallas as pl


def kernel(x, table):
    raise NotImplementedError("write your pallas kernel here")


# speedup vs baseline: 4.8564x; 4.8564x over previous
import jax
import jax.numpy as jnp
from jax import lax
from jax.experimental import pallas as pl
from jax.experimental.pallas import tpu as pltpu
from jax.experimental.pallas import tpu_sc as plsc

BLANK = 5
NC, NS, L = 2, 16, 16
NW = NC * NS

B, S, D = 4, 4096, 768
N = B * S
NPW = N // NW
CH = 64
NCHUNK = NPW // CH
NBUF = 2
WPR = S // NPW

HALO = 8
LX = HALO + NPW + 16
NT = D // L


def _body(x_hbm, table_hbm, out_hbm, lx, buf0, buf1, prow, g0, g1, s0, s1, psem):
    wid = lax.axis_index("s") * NC + lax.axis_index("c")
    base = wid * NPW
    row_start = (wid % WPR) == 0

    lx[pl.ds(0, 16)] = jnp.full((16,), BLANK, jnp.int32)
    lx[pl.ds(HALO + NPW, 16)] = jnp.zeros((16,), jnp.int32)
    pltpu.sync_copy(x_hbm.at[pl.ds(base, NPW)], lx.at[pl.ds(HALO, NPW)])

    @pl.when(jnp.logical_not(row_start))
    def _():
        pltpu.sync_copy(x_hbm.at[pl.ds(base - HALO, HALO)], lx.at[pl.ds(0, HALO)])

    def _gather(c, bufref, sem):
        return pltpu.make_async_copy(
            table_hbm.at[lx.at[pl.ds(HALO + c * CH, CH)]], bufref, sem)

    def _scatter(c, bufref, sem):
        return pltpu.make_async_copy(
            bufref, out_hbm.at[pl.ds(base + c * CH, CH)], sem)

    def _patch(c, bufb):
        start = c * CH
        any_blank = lx[pl.ds(start + 4, 16)] == BLANK
        for m in range(1, 5):
            any_blank = any_blank | (lx[pl.ds(start + 4 + m * 16, 16)] == BLANK)
        cnt = plsc.all_reduce_population_count(any_blank)[0]

        @pl.when(cnt > 0)
        def _():
            def qbody(qi, carry):
                q = start - 4 + qi
                pair = lx[pl.ds(q + HALO, 16)]
                a = pair[0]
                nxt = pair[1]

                @pl.when((nxt == BLANK) & (a != BLANK))
                def _():
                    idx = jnp.full((L,), a, jnp.int32)
                    pltpu.async_copy(table_hbm.at[idx], prow, psem).wait()
                    lo = jnp.maximum(1, start - q)
                    hi = jnp.minimum(4, start + CH - 1 - q) + 1

                    def kbody(k, carry2):
                        p = q + k - start

                        def tbody(t, carry3):
                            sl = pl.ds(t * L, L)
                            bufb[p, sl] = bufb[p, sl] + prow[0, sl]
                            return carry3

                        return lax.fori_loop(0, NT, tbody, carry2)

                    lax.fori_loop(lo, hi, kbody, jnp.int32(0))
                return carry

            lax.fori_loop(0, CH + 3, qbody, jnp.int32(0))

    bufs = (buf0, buf1)
    gsems = (g0, g1)
    ssems = (s0, s1)

    _gather(0, buf0, g0).start()
    _gather(1, buf1, g1).start()

    def group(gidx, carry):
        cbase = gidx * NBUF
        for b in range(NBUF):
            c = cbase + b
            _gather(c, bufs[b], gsems[b]).wait()
            _patch(c, bufs[b])
            _scatter(c, bufs[b], ssems[b]).start()

            @pl.when(c + NBUF < NCHUNK)
            def _():
                _scatter(c, bufs[b], ssems[b]).wait()
                _gather(c + NBUF, bufs[b], gsems[b]).start()
        return carry

    lax.fori_loop(0, NCHUNK // NBUF, group, jnp.int32(0))

    _scatter(NCHUNK - 2, buf0, s0).wait()
    _scatter(NCHUNK - 1, buf1, s1).wait()


def kernel(x, table):
    assert x.shape == (B, S) and table.shape[1] == D
    xf = x.reshape(N)
    mesh = plsc.VectorSubcoreMesh(core_axis_name="c", subcore_axis_name="s")
    out = pl.kernel(
        _body,
        out_type=jax.ShapeDtypeStruct((N, D), jnp.float32),
        mesh=mesh,
        compiler_params=pltpu.CompilerParams(needs_layout_passes=False),
        scratch_types=[
            pltpu.VMEM((LX,), jnp.int32),
            pltpu.VMEM((CH, D), jnp.float32),
            pltpu.VMEM((CH, D), jnp.float32),
            pltpu.VMEM((L, D), jnp.float32),
            pltpu.SemaphoreType.DMA,
            pltpu.SemaphoreType.DMA,
            pltpu.SemaphoreType.DMA,
            pltpu.SemaphoreType.DMA,
            pltpu.SemaphoreType.DMA,
        ],
    )(xf, table)
    return out.reshape(B, S, D)


# speedup vs baseline: 4.9307x; 1.0153x over previous
import jax
import jax.numpy as jnp
from jax import lax
from jax.experimental import pallas as pl
from jax.experimental.pallas import tpu as pltpu
from jax.experimental.pallas import tpu_sc as plsc

BLANK = 5
NC, NS, L = 2, 16, 16
NW = NC * NS

B, S, D = 4, 4096, 768
N = B * S
NPW = N // NW
CH = 32
NCHUNK = NPW // CH
NBUF = 4
PF = 2
WPR = S // NPW

HALO = 8
LX = HALO + NPW + 16
NT = D // L


def _body(x_hbm, table_hbm, out_hbm, lx, buf0, buf1, buf2, buf3, prow,
          g0, g1, g2, g3, s0, s1, s2, s3, psem):
    wid = lax.axis_index("s") * NC + lax.axis_index("c")
    base = wid * NPW
    row_start = (wid % WPR) == 0

    lx[pl.ds(0, 16)] = jnp.full((16,), BLANK, jnp.int32)
    lx[pl.ds(HALO + NPW, 16)] = jnp.zeros((16,), jnp.int32)
    pltpu.sync_copy(x_hbm.at[pl.ds(base, NPW)], lx.at[pl.ds(HALO, NPW)])

    @pl.when(jnp.logical_not(row_start))
    def _():
        pltpu.sync_copy(x_hbm.at[pl.ds(base - HALO, HALO)], lx.at[pl.ds(0, HALO)])

    def _gather(c, bufref, sem):
        return pltpu.make_async_copy(
            table_hbm.at[lx.at[pl.ds(HALO + c * CH, CH)]], bufref, sem)

    def _scatter(c, bufref, sem):
        return pltpu.make_async_copy(
            bufref, out_hbm.at[pl.ds(base + c * CH, CH)], sem)

    def _patch(c, bufb):
        start = c * CH
        any_blank = lx[pl.ds(start + 4, 16)] == BLANK
        for m in range(1, (CH + 16 + 15) // 16):
            any_blank = any_blank | (lx[pl.ds(start + 4 + m * 16, 16)] == BLANK)
        cnt = plsc.all_reduce_population_count(any_blank)[0]

        @pl.when(cnt > 0)
        def _():
            def qbody(qi, carry):
                q = start - 4 + qi
                pair = lx[pl.ds(q + HALO, 16)]
                a = pair[0]
                nxt = pair[1]

                @pl.when((nxt == BLANK) & (a != BLANK))
                def _():
                    idx = jnp.full((L,), a, jnp.int32)
                    pltpu.async_copy(table_hbm.at[idx], prow, psem).wait()
                    lo = jnp.maximum(1, start - q)
                    hi = jnp.minimum(4, start + CH - 1 - q) + 1

                    def kbody(k, carry2):
                        p = q + k - start

                        def tbody(t, carry3):
                            sl = pl.ds(t * L, L)
                            bufb[p, sl] = bufb[p, sl] + prow[0, sl]
                            return carry3

                        return lax.fori_loop(0, NT, tbody, carry2)

                    lax.fori_loop(lo, hi, kbody, jnp.int32(0))
                return carry

            lax.fori_loop(0, CH + 3, qbody, jnp.int32(0))

    bufs = (buf0, buf1, buf2, buf3)
    gsems = (g0, g1, g2, g3)
    ssems = (s0, s1, s2, s3)

    for c0 in range(PF):
        _gather(c0, bufs[c0], gsems[c0]).start()

    def group(gidx, carry):
        cbase = gidx * NBUF
        for b in range(NBUF):
            c = cbase + b
            _gather(c, bufs[b], gsems[b]).wait()
            _patch(c, bufs[b])
            _scatter(c, bufs[b], ssems[b]).start()

            bn = (b + PF) % NBUF

            @pl.when(c + PF < NCHUNK)
            def _():
                @pl.when(c + PF >= NBUF)
                def _():
                    _scatter(c + PF - NBUF, bufs[bn], ssems[bn]).wait()

                _gather(c + PF, bufs[bn], gsems[bn]).start()
        return carry

    lax.fori_loop(0, NCHUNK // NBUF, group, jnp.int32(0))

    for c0 in range(NCHUNK - NBUF, NCHUNK):
        b = c0 % NBUF
        _scatter(c0, bufs[b], ssems[b]).wait()


def kernel(x, table):
    assert x.shape == (B, S) and table.shape[1] == D
    xf = x.reshape(N)
    mesh = plsc.VectorSubcoreMesh(core_axis_name="c", subcore_axis_name="s")
    out = pl.kernel(
        _body,
        out_type=jax.ShapeDtypeStruct((N, D), jnp.float32),
        mesh=mesh,
        compiler_params=pltpu.CompilerParams(needs_layout_passes=False),
        scratch_types=(
            [pltpu.VMEM((LX,), jnp.int32)]
            + [pltpu.VMEM((CH, D), jnp.float32) for _ in range(NBUF)]
            + [pltpu.VMEM((L, D), jnp.float32)]
            + [pltpu.SemaphoreType.DMA for _ in range(2 * NBUF + 1)]
        ),
    )(xf, table)
    return out.reshape(B, S, D)
